# baseline (speedup 1.0000x reference)
import functools

import jax
import jax.numpy as jnp
from jax import lax
from jax.experimental import pallas as pl
from jax.experimental.pallas import tpu as pltpu
from jax.experimental.pallas import tpu_sc as plsc

C = 3
T = 300
S = 32
H, W = 256, 256
NC, NS_SUB = 2, 16
NW = NC * NS_SUB
FRAMES = C * S
FPW = FRAMES // NW
CHF = 4
RCH = H // CHF
CPW = FPW * CHF
NBUF = 6
RD = 3


def _sc_subsample(x):
    mesh = plsc.VectorSubcoreMesh(core_axis_name="c", subcore_axis_name="s")

    @functools.partial(
        pl.kernel,
        mesh=mesh,
        out_type=jax.ShapeDtypeStruct((C, S, H, W), jnp.float32),
        scratch_types=(
            [pltpu.VMEM((RCH, W), jnp.float32) for _ in range(NBUF)]
            + [pltpu.SemaphoreType.DMA((NBUF,)),
               pltpu.SemaphoreType.DMA((NBUF,))]
        ),
    )
    def k(x_hbm, out_hbm, *rest):
        bufs, (rsem, wsem) = rest[:NBUF], rest[NBUF:]
        wid = lax.axis_index("s") * NC + lax.axis_index("c")
        f0 = wid * FPW

        def coords(kk):
            f = f0 + (kk // CHF)
            c = f // S
            i = f - c * S
            src = (i * (T - 1)) // (S - 1)
            return c, i, src, (kk % CHF) * RCH

        def read(kk):
            c, _, src, r0 = coords(kk)
            return pltpu.async_copy(
                x_hbm.at[c, src, pl.ds(r0, RCH), :],
                bufs[kk % NBUF], rsem.at[kk % NBUF])

        def write(kk):
            c, i, _, r0 = coords(kk)
            return pltpu.async_copy(
                bufs[kk % NBUF],
                out_hbm.at[c, i, pl.ds(r0, RCH), :], wsem.at[kk % NBUF])

        reads = [None] * CPW
        writes = [None] * CPW
        wwaited = [False] * CPW
        for kk in range(min(RD, CPW)):
            reads[kk] = read(kk)
        for kk in range(CPW):
            reads[kk].wait()
            writes[kk] = write(kk)
            nxt = kk + RD
            if nxt < CPW:
                prev = nxt - NBUF
                if prev >= 0:
                    writes[prev].wait()
                    wwaited[prev] = True
                reads[nxt] = read(nxt)
        for kk in range(CPW):
            if not wwaited[kk]:
                writes[kk].wait()

    return k(x)


def kernel(x):
    return _sc_subsample(x)


# speedup vs baseline: 1.0091x; 1.0091x over previous
import functools

import jax
import jax.numpy as jnp
from jax import lax
from jax.experimental import pallas as pl
from jax.experimental.pallas import tpu as pltpu
from jax.experimental.pallas import tpu_sc as plsc

C = 3
T = 300
S = 32
H, W = 256, 256
NC, NS_SUB = 2, 16
NW = NC * NS_SUB
FRAMES = C * S
FPW = FRAMES // NW
CHF = 2
RCH = H // CHF
CPW = FPW * CHF
NBUF = 3
RD = 2


def _sc_subsample(x):
    mesh = plsc.VectorSubcoreMesh(core_axis_name="c", subcore_axis_name="s")

    @functools.partial(
        pl.kernel,
        mesh=mesh,
        out_type=jax.ShapeDtypeStruct((C, S, H, W), jnp.float32),
        scratch_types=(
            [pltpu.VMEM((RCH, W), jnp.float32) for _ in range(NBUF)]
            + [pltpu.SemaphoreType.DMA((NBUF,)),
               pltpu.SemaphoreType.DMA((NBUF,))]
        ),
    )
    def k(x_hbm, out_hbm, *rest):
        bufs, (rsem, wsem) = rest[:NBUF], rest[NBUF:]
        wid = lax.axis_index("s") * NC + lax.axis_index("c")
        f0 = wid * FPW

        def coords(kk):
            f = f0 + (kk // CHF)
            c = f // S
            i = f - c * S
            src = (i * (T - 1)) // (S - 1)
            return c, i, src, (kk % CHF) * RCH

        def read(kk):
            c, _, src, r0 = coords(kk)
            return pltpu.async_copy(
                x_hbm.at[c, src, pl.ds(r0, RCH), :],
                bufs[kk % NBUF], rsem.at[kk % NBUF])

        def write(kk):
            c, i, _, r0 = coords(kk)
            return pltpu.async_copy(
                bufs[kk % NBUF],
                out_hbm.at[c, i, pl.ds(r0, RCH), :], wsem.at[kk % NBUF])

        reads = [None] * CPW
        writes = [None] * CPW
        wwaited = [False] * CPW
        for kk in range(min(RD, CPW)):
            reads[kk] = read(kk)
        for kk in range(CPW):
            reads[kk].wait()
            writes[kk] = write(kk)
            nxt = kk + RD
            if nxt < CPW:
                prev = nxt - NBUF
                if prev >= 0:
                    writes[prev].wait()
                    wwaited[prev] = True
                reads[nxt] = read(nxt)
        for kk in range(CPW):
            if not wwaited[kk]:
                writes[kk].wait()

    return k(x)


def kernel(x):
    return _sc_subsample(x)
